# baseline (speedup 1.0000x reference)
import jax
import jax.numpy as jnp
from jax import lax
from jax.experimental import pallas as pl
from jax.experimental.pallas import tpu as pltpu

D_MODEL = 768
NUM_EXPERTS = 8
TOP_K = 2
EXPERT_DIM = 128
EH = NUM_EXPERTS * EXPERT_DIM
TOKEN_BLOCK = 2048


def _moe_block(x_ref, gw_ref, gb_ref, w1_ref, b1_ref, w2_ref, b2_ref, out_ref):
    xb = x_ref[...]
    w1a = jnp.concatenate(
        [w1_ref[e].astype(jnp.bfloat16) for e in range(NUM_EXPERTS)], axis=1)
    tb = xb.shape[0]

    logits = jnp.dot(xb, gw_ref[...], preferred_element_type=jnp.float32)
    logits = logits + gb_ref[...]

    iota = lax.broadcasted_iota(jnp.int32, (tb, NUM_EXPERTS), 1)
    big = jnp.int32(NUM_EXPERTS + 1)
    l1 = jnp.max(logits, axis=-1, keepdims=True)
    idx1 = jnp.min(jnp.where(logits >= l1, iota, big), axis=-1, keepdims=True)
    sel1 = iota == idx1
    lm = jnp.where(sel1, -jnp.inf, logits)
    l2 = jnp.max(lm, axis=-1, keepdims=True)
    idx2 = jnp.min(jnp.where(lm >= l2, iota, big), axis=-1, keepdims=True)
    sel2 = iota == idx2
    r = jnp.exp(l2 - l1)
    c1 = 1.0 / (1.0 + r)
    coef = jnp.where(sel1, c1, 0.0) + jnp.where(sel2, r * c1, 0.0)

    ei = lax.broadcasted_iota(jnp.int32, (NUM_EXPERTS, EH), 0)
    hi = lax.broadcasted_iota(jnp.int32, (NUM_EXPERTS, EH), 1)
    expand = (ei == (hi >> 7)).astype(jnp.float32)
    ce = jnp.dot(coef, expand, preferred_element_type=jnp.float32)

    h = jnp.dot(xb.astype(jnp.bfloat16), w1a,
                preferred_element_type=jnp.float32)
    h = jnp.maximum(h + b1_ref[...], 0.0)
    acc = jnp.dot((h * ce).astype(jnp.bfloat16),
                  w2_ref[...].astype(jnp.bfloat16),
                  preferred_element_type=jnp.float32)
    acc = acc + jnp.dot(coef, b2_ref[...], preferred_element_type=jnp.float32)
    out_ref[...] = acc


def kernel(x, gate_W, gate_b, W1, b1, W2, b2):
    batch, seq, d = x.shape
    x_flat = x.reshape(-1, d)
    t = x_flat.shape[0]
    b1a = b1.reshape(1, EH)
    w2a = W2.reshape(EH, D_MODEL)
    grid = (t // TOKEN_BLOCK,)
    out = pl.pallas_call(
        _moe_block,
        grid=grid,
        in_specs=[
            pl.BlockSpec((TOKEN_BLOCK, D_MODEL), lambda i: (i, 0)),
            pl.BlockSpec((D_MODEL, NUM_EXPERTS), lambda i: (0, 0)),
            pl.BlockSpec((1, NUM_EXPERTS), lambda i: (0, 0)),
            pl.BlockSpec((NUM_EXPERTS, D_MODEL, EXPERT_DIM), lambda i: (0, 0, 0)),
            pl.BlockSpec((1, EH), lambda i: (0, 0)),
            pl.BlockSpec((EH, D_MODEL), lambda i: (0, 0)),
            pl.BlockSpec((NUM_EXPERTS, D_MODEL), lambda i: (0, 0)),
        ],
        out_specs=pl.BlockSpec((TOKEN_BLOCK, D_MODEL), lambda i: (i, 0)),
        out_shape=jax.ShapeDtypeStruct((t, D_MODEL), jnp.float32),
        compiler_params=pltpu.CompilerParams(
            dimension_semantics=("parallel",),
        ),
    )(x_flat, gate_W, gate_b.reshape(1, -1), W1, b1a, w2a, b2)
    return out.reshape(batch, seq, d)


# speedup vs baseline: 1.0605x; 1.0605x over previous
import jax
import jax.numpy as jnp
from jax import lax
from jax.experimental import pallas as pl
from jax.experimental.pallas import tpu as pltpu

D_MODEL = 768
NUM_EXPERTS = 8
TOP_K = 2
EXPERT_DIM = 128
EH = NUM_EXPERTS * EXPERT_DIM
TOKEN_BLOCK = 1024


def _moe_block(x_ref, gw_ref, gb_ref, w1_ref, b1_ref, w2_ref, b2_ref, out_ref):
    xb = x_ref[...]
    w1a = jnp.concatenate(
        [w1_ref[e].astype(jnp.bfloat16) for e in range(NUM_EXPERTS)], axis=1)
    tb = xb.shape[0]

    logits = jnp.dot(xb, gw_ref[...], preferred_element_type=jnp.float32)
    logits = logits + gb_ref[...]

    iota = lax.broadcasted_iota(jnp.int32, (tb, NUM_EXPERTS), 1)
    big = jnp.int32(NUM_EXPERTS + 1)
    l1 = jnp.max(logits, axis=-1, keepdims=True)
    idx1 = jnp.min(jnp.where(logits >= l1, iota, big), axis=-1, keepdims=True)
    sel1 = iota == idx1
    lm = jnp.where(sel1, -jnp.inf, logits)
    l2 = jnp.max(lm, axis=-1, keepdims=True)
    idx2 = jnp.min(jnp.where(lm >= l2, iota, big), axis=-1, keepdims=True)
    sel2 = iota == idx2
    r = jnp.exp(l2 - l1)
    c1 = 1.0 / (1.0 + r)
    coef = jnp.where(sel1, c1, 0.0) + jnp.where(sel2, r * c1, 0.0)

    ei = lax.broadcasted_iota(jnp.int32, (NUM_EXPERTS, EH), 0)
    hi = lax.broadcasted_iota(jnp.int32, (NUM_EXPERTS, EH), 1)
    expand = (ei == (hi >> 7)).astype(jnp.float32)
    ce = jnp.dot(coef, expand, preferred_element_type=jnp.float32)

    h = jnp.dot(xb.astype(jnp.bfloat16), w1a,
                preferred_element_type=jnp.float32)
    h = jnp.maximum(h + b1_ref[...], 0.0)
    acc = jnp.dot((h * ce).astype(jnp.bfloat16),
                  w2_ref[...].astype(jnp.bfloat16),
                  preferred_element_type=jnp.float32)
    acc = acc + jnp.dot(coef, b2_ref[...], preferred_element_type=jnp.float32)
    out_ref[...] = acc


def kernel(x, gate_W, gate_b, W1, b1, W2, b2):
    batch, seq, d = x.shape
    x_flat = x.reshape(-1, d)
    t = x_flat.shape[0]
    b1a = b1.reshape(1, EH)
    w2a = W2.reshape(EH, D_MODEL)
    grid = (t // TOKEN_BLOCK,)
    out = pl.pallas_call(
        _moe_block,
        grid=grid,
        in_specs=[
            pl.BlockSpec((TOKEN_BLOCK, D_MODEL), lambda i: (i, 0)),
            pl.BlockSpec((D_MODEL, NUM_EXPERTS), lambda i: (0, 0)),
            pl.BlockSpec((1, NUM_EXPERTS), lambda i: (0, 0)),
            pl.BlockSpec((NUM_EXPERTS, D_MODEL, EXPERT_DIM), lambda i: (0, 0, 0)),
            pl.BlockSpec((1, EH), lambda i: (0, 0)),
            pl.BlockSpec((EH, D_MODEL), lambda i: (0, 0)),
            pl.BlockSpec((NUM_EXPERTS, D_MODEL), lambda i: (0, 0)),
        ],
        out_specs=pl.BlockSpec((TOKEN_BLOCK, D_MODEL), lambda i: (i, 0)),
        out_shape=jax.ShapeDtypeStruct((t, D_MODEL), jnp.float32),
        compiler_params=pltpu.CompilerParams(
            dimension_semantics=("parallel",),
        ),
    )(x_flat, gate_W, gate_b.reshape(1, -1), W1, b1a, w2a, b2)
    return out.reshape(batch, seq, d)
